# baseline (speedup 1.0000x reference)
import functools

import jax
import jax.numpy as jnp
from jax.experimental import pallas as pl
from jax.experimental.pallas import tpu as pltpu

N = 512
ND = 128
ED = 16
H = 64
ROUNDS = 3
BI = 128
NBI = N // BI


BT = 32


def _tr_body(ef_ref, adj_ref, out_ref):
    y = jnp.transpose(ef_ref[:], (0, 2, 1))
    out_ref[:, :ED, :] = y.astype(jnp.bfloat16)
    out_ref[:, ED, :] = jnp.ones((BT, N), jnp.bfloat16)
    madd = ((adj_ref[:] != 0).astype(jnp.float32) - 1.0) * 1e30
    out_ref[:, ED + 1, :] = madd.astype(jnp.bfloat16)


def _transpose_aug(edge_features, adjacency):
    return pl.pallas_call(
        _tr_body,
        grid=(N // BT,),
        in_specs=[
            pl.BlockSpec((BT, N, ED), lambda i: (i, 0, 0)),
            pl.BlockSpec((BT, N), lambda i: (i, 0)),
        ],
        out_specs=pl.BlockSpec((BT, ED + 2, N), lambda i: (i, 0, 0)),
        out_shape=jax.ShapeDtypeStruct((N, ED + 2, N), jnp.bfloat16),
    )(edge_features, adjacency)


def _body(ef_ref, adj_ref, nf_ref, Wn_ref, bn_ref, W1a_ref, W1c_ref,
          Wft_ref, c_ref, W2_ref, b2_ref,
          Wih_ref, bih_ref, Whh_ref, bhh_ref,
          out_ref, nodes_s, bt_s):
    r = pl.program_id(0)
    ib = pl.program_id(1)

    @pl.when(jnp.logical_and(r == 0, ib == 0))
    def _init():
        nodes_s[:] = (
            jnp.dot(nf_ref[:], Wn_ref[:], preferred_element_type=jnp.float32)
            + bn_ref[:]
        )

    @pl.when(ib == 0)
    def _round_start():
        bt_s[:] = jax.lax.dot_general(
            W1c_ref[:], nodes_s[:],
            dimension_numbers=(((0,), (1,)), ((), ())),
            preferred_element_type=jnp.float32,
        )

    old = nodes_s[pl.ds(ib * BI, BI), :]
    a_blk = (
        jnp.dot(old, W1a_ref[:], preferred_element_type=jnp.float32)
        + c_ref[:]
    )

    lhs = jnp.concatenate(
        [
            jnp.broadcast_to(Wft_ref[:][None], (BI, H, ED)),
            a_blk.astype(jnp.bfloat16)[:, :, None],
            jnp.ones((BI, H, 1), jnp.bfloat16),
        ],
        axis=2,
    )
    pre = jax.lax.dot_general(
        lhs, ef_ref[:],
        dimension_numbers=(((2,), (1,)), ((0,), (0,))),
        preferred_element_type=jnp.float32,
    )
    s = jnp.sum(jnp.maximum(pre + bt_s[:][None, :, :], 0.0), axis=2)

    mask = (adj_ref[:] != 0).astype(jnp.float32)
    deg = jnp.sum(mask, axis=1, keepdims=True)
    safe = jnp.maximum(deg, 1.0)
    m = jnp.dot(s, W2_ref[:], preferred_element_type=jnp.float32) / safe + b2_ref[:]
    m = jnp.where(deg > 0, m, 0.0)

    gi = jnp.dot(m, Wih_ref[:], preferred_element_type=jnp.float32) + bih_ref[:]
    gh = jnp.dot(old, Whh_ref[:], preferred_element_type=jnp.float32) + bhh_ref[:]
    r_g = jax.nn.sigmoid(gi[:, :H] + gh[:, :H])
    z_g = jax.nn.sigmoid(gi[:, H:2 * H] + gh[:, H:2 * H])
    ng = jnp.tanh(gi[:, 2 * H:] + r_g * gh[:, 2 * H:])
    new = (1.0 - z_g) * ng + z_g * old

    nodes_s[pl.ds(ib * BI, BI), :] = new
    out_ref[:] = new


@jax.jit
def kernel(node_features, edge_features, adjacency, Wn, bn, We, be,
           W1, b1, W2, b2, Wih, bih, Whh, bhh):
    W1a = W1[:H]
    W1e = W1[H:2 * H]
    W1c = W1[2 * H:]
    wft = (We @ W1e).T.astype(jnp.bfloat16)
    c = (b1 + be @ W1e).reshape(1, H)
    ef_aug = _transpose_aug(edge_features, adjacency)

    grid = (ROUNDS, NBI)
    out = pl.pallas_call(
        _body,
        grid=grid,
        in_specs=[
            pl.BlockSpec((BI, ED + 2, N), lambda r, ib: (ib, 0, 0)),
            pl.BlockSpec((BI, N), lambda r, ib: (ib, 0)),
            pl.BlockSpec((N, ND), lambda r, ib: (0, 0)),
            pl.BlockSpec((ND, H), lambda r, ib: (0, 0)),
            pl.BlockSpec((1, H), lambda r, ib: (0, 0)),
            pl.BlockSpec((H, H), lambda r, ib: (0, 0)),
            pl.BlockSpec((H, H), lambda r, ib: (0, 0)),
            pl.BlockSpec((H, ED), lambda r, ib: (0, 0)),
            pl.BlockSpec((1, H), lambda r, ib: (0, 0)),
            pl.BlockSpec((H, H), lambda r, ib: (0, 0)),
            pl.BlockSpec((1, H), lambda r, ib: (0, 0)),
            pl.BlockSpec((H, 3 * H), lambda r, ib: (0, 0)),
            pl.BlockSpec((1, 3 * H), lambda r, ib: (0, 0)),
            pl.BlockSpec((H, 3 * H), lambda r, ib: (0, 0)),
            pl.BlockSpec((1, 3 * H), lambda r, ib: (0, 0)),
        ],
        out_specs=pl.BlockSpec((BI, H), lambda r, ib: (ib, 0)),
        out_shape=jax.ShapeDtypeStruct((N, H), jnp.float32),
        scratch_shapes=[
            pltpu.VMEM((N, H), jnp.float32),
            pltpu.VMEM((H, N), jnp.float32),
        ],
    )(
        ef_aug, adjacency, node_features, Wn, bn.reshape(1, H), W1a, W1c,
        wft, c, W2, b2.reshape(1, H),
        Wih, bih.reshape(1, 3 * H), Whh, bhh.reshape(1, 3 * H),
    )
    return out


# speedup vs baseline: 2.5533x; 2.5533x over previous
import jax
import jax.numpy as jnp
from jax.experimental import pallas as pl
from jax.experimental.pallas import tpu as pltpu

N = 512
ND = 128
ED = 16
H = 64
ROUNDS = 3
BI = 128
NBI = N // BI


def _body(ef_ref, adj_ref, nf_ref, Wn_ref, bn_ref, W1a_ref, W1c_ref,
          Wft_ref, c_ref, W2_ref, b2_ref,
          Wih_ref, bih_ref, Whh_ref, bhh_ref,
          out_ref, nodes_s, bt_s):
    r = pl.program_id(0)
    ib = pl.program_id(1)

    @pl.when(jnp.logical_and(r == 0, ib == 0))
    def _init():
        nodes_s[:] = (
            jnp.dot(nf_ref[:], Wn_ref[:], preferred_element_type=jnp.float32)
            + bn_ref[:]
        )

    @pl.when(ib == 0)
    def _round_start():
        bt_s[:] = jax.lax.dot_general(
            W1c_ref[:], nodes_s[:],
            dimension_numbers=(((0,), (1,)), ((), ())),
            preferred_element_type=jnp.float32,
        )

    old = nodes_s[pl.ds(ib * BI, BI), :]
    a_blk = (
        jnp.dot(old, W1a_ref[:], preferred_element_type=jnp.float32)
        + c_ref[:]
    )

    lhs = jnp.concatenate(
        [
            jnp.broadcast_to(Wft_ref[:][None], (BI, H, ED)),
            a_blk.astype(jnp.bfloat16)[:, :, None],
            jnp.ones((BI, H, 1), jnp.bfloat16),
        ],
        axis=2,
    )
    pre = jax.lax.dot_general(
        lhs, ef_ref[:],
        dimension_numbers=(((2,), (1,)), ((0,), (0,))),
        preferred_element_type=jnp.float32,
    )
    s = jnp.sum(jnp.maximum(pre + bt_s[:][None, :, :], 0.0), axis=2)

    mask = (adj_ref[:] != 0).astype(jnp.float32)
    deg = jnp.sum(mask, axis=1, keepdims=True)
    safe = jnp.maximum(deg, 1.0)
    m = jnp.dot(s, W2_ref[:], preferred_element_type=jnp.float32) / safe + b2_ref[:]
    m = jnp.where(deg > 0, m, 0.0)

    gi = jnp.dot(m, Wih_ref[:], preferred_element_type=jnp.float32) + bih_ref[:]
    gh = jnp.dot(old, Whh_ref[:], preferred_element_type=jnp.float32) + bhh_ref[:]
    r_g = jax.nn.sigmoid(gi[:, :H] + gh[:, :H])
    z_g = jax.nn.sigmoid(gi[:, H:2 * H] + gh[:, H:2 * H])
    ng = jnp.tanh(gi[:, 2 * H:] + r_g * gh[:, 2 * H:])
    new = (1.0 - z_g) * ng + z_g * old

    nodes_s[pl.ds(ib * BI, BI), :] = new
    out_ref[:] = new


@jax.jit
def kernel(node_features, edge_features, adjacency, Wn, bn, We, be,
           W1, b1, W2, b2, Wih, bih, Whh, bhh):
    W1a = W1[:H]
    W1e = W1[H:2 * H]
    W1c = W1[2 * H:]
    wft = (We @ W1e).T.astype(jnp.bfloat16)
    c = (b1 + be @ W1e).reshape(1, H)
    mask_f = (adjacency != 0).astype(jnp.float32)
    ef_big = jnp.concatenate(
        [
            edge_features,
            jnp.ones((N, N, 1), jnp.float32),
            ((mask_f - 1.0) * 1e30)[:, :, None],
        ],
        axis=2,
    )
    ef_aug = jnp.transpose(ef_big, (0, 2, 1)).astype(jnp.bfloat16)

    grid = (ROUNDS, NBI)
    out = pl.pallas_call(
        _body,
        grid=grid,
        in_specs=[
            pl.BlockSpec((BI, ED + 2, N), lambda r, ib: (ib, 0, 0)),
            pl.BlockSpec((BI, N), lambda r, ib: (ib, 0)),
            pl.BlockSpec((N, ND), lambda r, ib: (0, 0)),
            pl.BlockSpec((ND, H), lambda r, ib: (0, 0)),
            pl.BlockSpec((1, H), lambda r, ib: (0, 0)),
            pl.BlockSpec((H, H), lambda r, ib: (0, 0)),
            pl.BlockSpec((H, H), lambda r, ib: (0, 0)),
            pl.BlockSpec((H, ED), lambda r, ib: (0, 0)),
            pl.BlockSpec((1, H), lambda r, ib: (0, 0)),
            pl.BlockSpec((H, H), lambda r, ib: (0, 0)),
            pl.BlockSpec((1, H), lambda r, ib: (0, 0)),
            pl.BlockSpec((H, 3 * H), lambda r, ib: (0, 0)),
            pl.BlockSpec((1, 3 * H), lambda r, ib: (0, 0)),
            pl.BlockSpec((H, 3 * H), lambda r, ib: (0, 0)),
            pl.BlockSpec((1, 3 * H), lambda r, ib: (0, 0)),
        ],
        out_specs=pl.BlockSpec((BI, H), lambda r, ib: (ib, 0)),
        out_shape=jax.ShapeDtypeStruct((N, H), jnp.float32),
        scratch_shapes=[
            pltpu.VMEM((N, H), jnp.float32),
            pltpu.VMEM((H, N), jnp.float32),
        ],
    )(
        ef_aug, adjacency, node_features, Wn, bn.reshape(1, H), W1a, W1c,
        wft, c, W2, b2.reshape(1, H),
        Wih, bih.reshape(1, 3 * H), Whh, bhh.reshape(1, 3 * H),
    )
    return out


# speedup vs baseline: 2.5590x; 1.0022x over previous
import jax
import jax.numpy as jnp
from jax.experimental import pallas as pl
from jax.experimental.pallas import tpu as pltpu

N = 512
ND = 128
ED = 16
H = 64
ROUNDS = 3
BI = 128
NBI = N // BI


def _body(ef_ref, adj_ref, nf_ref, Wn_ref, bn_ref, W1a_ref, W1c_ref,
          Wft_ref, c_ref, W2_ref, b2_ref,
          Wih_ref, bih_ref, Whh_ref, bhh_ref,
          out_ref, nodes_s, bt_s):
    r = pl.program_id(0)
    ib = pl.program_id(1)

    @pl.when(jnp.logical_and(r == 0, ib == 0))
    def _init():
        nodes_s[:] = (
            jnp.dot(nf_ref[:], Wn_ref[:], preferred_element_type=jnp.float32)
            + bn_ref[:]
        )

    @pl.when(ib == 0)
    def _round_start():
        bt_s[:] = jax.lax.dot_general(
            W1c_ref[:], nodes_s[:],
            dimension_numbers=(((0,), (1,)), ((), ())),
            preferred_element_type=jnp.float32,
        )

    old = nodes_s[pl.ds(ib * BI, BI), :]
    a_blk = (
        jnp.dot(old, W1a_ref[:], preferred_element_type=jnp.float32)
        + c_ref[:]
    )

    lhs = jnp.concatenate(
        [
            jnp.broadcast_to(Wft_ref[:][None], (BI, H, ED)),
            a_blk.astype(jnp.bfloat16)[:, :, None],
            jnp.ones((BI, H, 1), jnp.bfloat16),
        ],
        axis=2,
    )
    pre = jax.lax.dot_general(
        lhs, ef_ref[:],
        dimension_numbers=(((2,), (1,)), ((0,), (0,))),
        preferred_element_type=jnp.float32,
    )
    s = jnp.sum(jnp.maximum(pre + bt_s[:][None, :, :], 0.0), axis=2)

    mask = (adj_ref[:] != 0).astype(jnp.float32)
    deg = jnp.sum(mask, axis=1, keepdims=True)
    safe = jnp.maximum(deg, 1.0)
    m = jnp.dot(s, W2_ref[:], preferred_element_type=jnp.float32) / safe + b2_ref[:]
    m = jnp.where(deg > 0, m, 0.0)

    gi = jnp.dot(m, Wih_ref[:], preferred_element_type=jnp.float32) + bih_ref[:]
    gh = jnp.dot(old, Whh_ref[:], preferred_element_type=jnp.float32) + bhh_ref[:]
    r_g = jax.nn.sigmoid(gi[:, :H] + gh[:, :H])
    z_g = jax.nn.sigmoid(gi[:, H:2 * H] + gh[:, H:2 * H])
    ng = jnp.tanh(gi[:, 2 * H:] + r_g * gh[:, 2 * H:])
    new = (1.0 - z_g) * ng + z_g * old

    nodes_s[pl.ds(ib * BI, BI), :] = new
    out_ref[:] = new


@jax.jit
def kernel(node_features, edge_features, adjacency, Wn, bn, We, be,
           W1, b1, W2, b2, Wih, bih, Whh, bhh):
    W1a = W1[:H]
    W1e = W1[H:2 * H]
    W1c = W1[2 * H:]
    wft = (We @ W1e).T.astype(jnp.bfloat16)
    c = (b1 + be @ W1e).reshape(1, H)
    mask_f = (adjacency != 0).astype(jnp.float32)
    ef_bf = jax.lax.optimization_barrier(edge_features.astype(jnp.bfloat16))
    ef_big = jnp.concatenate(
        [
            ef_bf,
            jnp.ones((N, N, 1), jnp.bfloat16),
            ((mask_f - 1.0) * 1e30)[:, :, None].astype(jnp.bfloat16),
        ],
        axis=2,
    )
    ef_aug = jnp.transpose(ef_big, (0, 2, 1))

    grid = (ROUNDS, NBI)
    out = pl.pallas_call(
        _body,
        grid=grid,
        in_specs=[
            pl.BlockSpec((BI, ED + 2, N), lambda r, ib: (ib, 0, 0)),
            pl.BlockSpec((BI, N), lambda r, ib: (ib, 0)),
            pl.BlockSpec((N, ND), lambda r, ib: (0, 0)),
            pl.BlockSpec((ND, H), lambda r, ib: (0, 0)),
            pl.BlockSpec((1, H), lambda r, ib: (0, 0)),
            pl.BlockSpec((H, H), lambda r, ib: (0, 0)),
            pl.BlockSpec((H, H), lambda r, ib: (0, 0)),
            pl.BlockSpec((H, ED), lambda r, ib: (0, 0)),
            pl.BlockSpec((1, H), lambda r, ib: (0, 0)),
            pl.BlockSpec((H, H), lambda r, ib: (0, 0)),
            pl.BlockSpec((1, H), lambda r, ib: (0, 0)),
            pl.BlockSpec((H, 3 * H), lambda r, ib: (0, 0)),
            pl.BlockSpec((1, 3 * H), lambda r, ib: (0, 0)),
            pl.BlockSpec((H, 3 * H), lambda r, ib: (0, 0)),
            pl.BlockSpec((1, 3 * H), lambda r, ib: (0, 0)),
        ],
        out_specs=pl.BlockSpec((BI, H), lambda r, ib: (ib, 0)),
        out_shape=jax.ShapeDtypeStruct((N, H), jnp.float32),
        scratch_shapes=[
            pltpu.VMEM((N, H), jnp.float32),
            pltpu.VMEM((H, N), jnp.float32),
        ],
    )(
        ef_aug, adjacency, node_features, Wn, bn.reshape(1, H), W1a, W1c,
        wft, c, W2, b2.reshape(1, H),
        Wih, bih.reshape(1, 3 * H), Whh, bhh.reshape(1, 3 * H),
    )
    return out
